# baseline (speedup 1.0000x reference)
import functools

import jax
import jax.numpy as jnp
from jax.experimental import pallas as pl
from jax.experimental.pallas import tpu as pltpu


_ROWS = 512
_NSUB = 4
_LN2 = 0.6931471805599453


def _body(y_ref, x_hbm, o_ref, buf_ref, sem, *, B, C, scale):
    nchunks = B // _ROWS
    sub = _ROWS // _NSUB

    def copies(k):
        b = k % 2
        return [
            pltpu.make_async_copy(
                x_hbm.at[pl.ds(k * _ROWS + j * sub, sub), :],
                buf_ref.at[b, pl.ds(j * sub, sub), :],
                sem.at[b, j],
            )
            for j in range(_NSUB)
        ]

    for c in copies(0):
        c.start()
    for c in copies(1):
        c.start()

    acc = jnp.zeros((8, C), jnp.float32)
    for k in range(nchunks):
        b = k % 2
        for c in copies(k):
            c.wait()
        lg = jnp.log2(buf_ref[b])
        yb = y_ref[pl.ds(k * _ROWS, _ROWS), :]
        cols = jax.lax.broadcasted_iota(jnp.int32, lg.shape, 1)
        m = jnp.where(cols == yb, 0.0, lg)
        acc = acc + jnp.sum(m.reshape(_ROWS // 8, 8, C), axis=0)
        if k + 2 < nchunks:
            for c in copies(k + 2):
                c.start()

    o_ref[...] = jnp.sum(acc).reshape(1, 1) * scale


def kernel(y_pred, y):
    B, C = y_pred.shape
    scale = -_LN2 / ((C - 1) * B)
    y2 = y.reshape(B, 1).astype(jnp.int32)

    out = pl.pallas_call(
        functools.partial(_body, B=B, C=C, scale=scale),
        grid=(1,),
        in_specs=[
            pl.BlockSpec((B, 1), lambda i: (0, 0)),
            pl.BlockSpec(memory_space=pl.ANY),
        ],
        out_specs=pl.BlockSpec((1, 1), lambda i: (0, 0)),
        out_shape=jax.ShapeDtypeStruct((1, 1), jnp.float32),
        scratch_shapes=[
            pltpu.VMEM((2, _ROWS, C), jnp.float32),
            pltpu.SemaphoreType.DMA((2, _NSUB)),
        ],
    )(y2, y_pred)
    return out[0, 0]


# speedup vs baseline: 1.0541x; 1.0541x over previous
import functools

import jax
import jax.numpy as jnp
from jax.experimental import pallas as pl
from jax.experimental.pallas import tpu as pltpu


_BLOCK_B = 128
_NSTREAMS = 4


def _body(*refs, nsteps, scale):
    i = pl.program_id(0)
    ns = _NSTREAMS
    y_refs = refs[:ns]
    x_refs = refs[ns : 2 * ns]
    o_ref = refs[2 * ns]
    acc_ref = refs[2 * ns + 1]

    part = None
    for y_ref, x_ref in zip(y_refs, x_refs):
        x = x_ref[...]
        lg = jnp.log(x)
        cols = jax.lax.broadcasted_iota(jnp.int32, x.shape, 1)
        m = jnp.where(cols == y_ref[...], 0.0, lg)
        p = jnp.sum(m.reshape(m.shape[0] // 8, 8, m.shape[1]), axis=0)
        part = p if part is None else part + p

    @pl.when(i == 0)
    def _():
        acc_ref[...] = jnp.zeros_like(acc_ref)

    acc_ref[...] += part

    @pl.when(i == nsteps - 1)
    def _():
        o_ref[...] = jnp.sum(acc_ref[...]).reshape(1, 1) * scale


def kernel(y_pred, y):
    B, C = y_pred.shape
    bb = _BLOCK_B
    ns = _NSTREAMS
    nsteps = B // (bb * ns)
    scale = -1.0 / ((C - 1) * B)
    y2 = y.reshape(B, 1).astype(jnp.int32)

    def x_spec(s):
        return pl.BlockSpec((bb, C), lambda i, s=s: (i + s * nsteps, 0))

    def y_spec(s):
        return pl.BlockSpec((bb, 1), lambda i, s=s: (i + s * nsteps, 0))

    out = pl.pallas_call(
        functools.partial(_body, nsteps=nsteps, scale=scale),
        grid=(nsteps,),
        in_specs=[y_spec(s) for s in range(ns)] + [x_spec(s) for s in range(ns)],
        out_specs=pl.BlockSpec((1, 1), lambda i: (0, 0)),
        out_shape=jax.ShapeDtypeStruct((1, 1), jnp.float32),
        scratch_shapes=[pltpu.VMEM((8, C), jnp.float32)],
    )(*([y2] * ns + [y_pred] * ns))
    return out[0, 0]


# speedup vs baseline: 1.1133x; 1.0561x over previous
import functools

import jax
import jax.numpy as jnp
from jax.experimental import pallas as pl
from jax.experimental.pallas import tpu as pltpu


_BLOCK_B = 256
_NSTREAMS = 4


def _body(*refs, nsteps, scale):
    i = pl.program_id(0)
    ns = _NSTREAMS
    y_refs = refs[:ns]
    x_refs = refs[ns : 2 * ns]
    o_ref = refs[2 * ns]
    acc_ref = refs[2 * ns + 1]

    part = None
    for y_ref, x_ref in zip(y_refs, x_refs):
        x = x_ref[...]
        lg = jnp.log(x)
        cols = jax.lax.broadcasted_iota(jnp.int32, x.shape, 1)
        m = jnp.where(cols == y_ref[...], 0.0, lg)
        p = jnp.sum(m.reshape(m.shape[0] // 8, 8, m.shape[1]), axis=0)
        part = p if part is None else part + p

    @pl.when(i == 0)
    def _():
        acc_ref[...] = jnp.zeros_like(acc_ref)

    acc_ref[...] += part

    @pl.when(i == nsteps - 1)
    def _():
        o_ref[...] = jnp.sum(acc_ref[...]).reshape(1, 1) * scale


def kernel(y_pred, y):
    B, C = y_pred.shape
    bb = _BLOCK_B
    ns = _NSTREAMS
    nsteps = B // (bb * ns)
    scale = -1.0 / ((C - 1) * B)
    y2 = y.reshape(B, 1).astype(jnp.int32)

    def x_spec(s):
        return pl.BlockSpec((bb, C), lambda i, s=s: (i + s * nsteps, 0))

    def y_spec(s):
        return pl.BlockSpec((bb, 1), lambda i, s=s: (i + s * nsteps, 0))

    out = pl.pallas_call(
        functools.partial(_body, nsteps=nsteps, scale=scale),
        grid=(nsteps,),
        in_specs=[y_spec(s) for s in range(ns)] + [x_spec(s) for s in range(ns)],
        out_specs=pl.BlockSpec((1, 1), lambda i: (0, 0)),
        out_shape=jax.ShapeDtypeStruct((1, 1), jnp.float32),
        scratch_shapes=[pltpu.VMEM((8, C), jnp.float32)],
    )(*([y2] * ns + [y_pred] * ns))
    return out[0, 0]
